# baseline (speedup 1.0000x reference)
import functools

import jax
import jax.numpy as jnp
from jax import lax
from jax.experimental import pallas as pl
from jax.experimental.pallas import tpu as pltpu
from jax.experimental.pallas import tpu_sc as plsc

N = 10000
E = 320000
D = 128
DH = 64
DE = 4
DEP = 16

NC = 2
NS = 16
NW = NC * NS
CHUNK = 96
CPT = 106
E_PAD = NW * CPT * CHUNK
N_ACC = 10240
RPT = N_ACC // NS
ZB = 80


def _sc_segsum(x2, src2d, dst2d, ea2d):
    mesh = plsc.VectorSubcoreMesh(core_axis_name="c", subcore_axis_name="s")

    slot_scratch = [
        pltpu.VMEM((1, CHUNK), jnp.int32),
        pltpu.VMEM((1, CHUNK), jnp.int32),
        pltpu.VMEM((1, CHUNK), jnp.int32),
        pltpu.VMEM((1, CHUNK), jnp.int32),
        pltpu.VMEM((CHUNK, DH), jnp.float32),
        pltpu.VMEM((CHUNK, DH), jnp.float32),
        pltpu.VMEM((CHUNK, DEP), jnp.float32),
        pltpu.SemaphoreType.DMA,
        pltpu.SemaphoreType.DMA,
        pltpu.SemaphoreType.DMA,
    ]

    @functools.partial(
        pl.kernel,
        out_type=[
            jax.ShapeDtypeStruct((NC * N_ACC, DH), jnp.float32),
            jax.ShapeDtypeStruct((NC * N_ACC, DH), jnp.float32),
            jax.ShapeDtypeStruct((NC * N_ACC, DEP), jnp.float32),
        ],
        mesh=mesh,
        compiler_params=pltpu.CompilerParams(use_tc_tiling_on_sc=False),
        scratch_types=slot_scratch + slot_scratch + [
            pltpu.VMEM_SHARED((N_ACC, DH), jnp.float32),
            pltpu.VMEM_SHARED((N_ACC, DH), jnp.float32),
            pltpu.VMEM_SHARED((N_ACC, DEP), jnp.float32),
        ],
    )
    def k(x_hbm, src_hbm, dst_hbm, ea_hbm, outl_hbm, outr_hbm, oute_hbm,
          *refs):
        src_c = (refs[0], refs[10])
        dst_c = (refs[1], refs[11])
        idxl = (refs[2], refs[12])
        idxr = (refs[3], refs[13])
        rl = (refs[4], refs[14])
        rr = (refs[5], refs[15])
        ea_v = (refs[6], refs[16])
        sem_g = (refs[7], refs[17])
        sem_e = (refs[8], refs[18])
        sem_s = (refs[9], refs[19])
        accl_sh, accr_sh, acce_sh = refs[20], refs[21], refs[22]

        cid = lax.axis_index("c")
        sid = lax.axis_index("s")
        wid = cid * NS + sid

        zero16 = jnp.zeros((16,), jnp.float32)

        def zx(i, _):
            rl[0][i // (DH // 16), pl.ds((i % (DH // 16)) * 16, 16)] = zero16
            return 0
        lax.fori_loop(0, ZB * (DH // 16), zx, 0)

        def ze(i, _):
            ea_v[0][i, pl.ds(0, 16)] = zero16
            return 0
        lax.fori_loop(0, ZB, ze, 0)

        @pl.loop(0, RPT // ZB)
        def zcopy(i):
            r = sid * RPT + i * ZB
            pltpu.sync_copy(rl[0].at[pl.ds(0, ZB)], accl_sh.at[pl.ds(r, ZB)])
            pltpu.sync_copy(rl[0].at[pl.ds(0, ZB)], accr_sh.at[pl.ds(r, ZB)])
            pltpu.sync_copy(ea_v[0].at[pl.ds(0, ZB)], acce_sh.at[pl.ds(r, ZB)])

        plsc.subcore_barrier()

        base = wid * CPT

        @pl.loop(0, CPT // 2)
        def piter(p):
            c0 = base + 2 * p
            gathers = []
            for b in range(2):
                c = c0 + b
                pltpu.sync_copy(src_hbm.at[pl.ds(c, 1)], src_c[b])
                pltpu.sync_copy(dst_hbm.at[pl.ds(c, 1)], dst_c[b])
                for t in range(CHUNK // 16):
                    s = src_c[b][0, pl.ds(t * 16, 16)]
                    s2 = s + s
                    idxl[b][0, pl.ds(t * 16, 16)] = s2
                    idxr[b][0, pl.ds(t * 16, 16)] = s2 + 1
                gathers.append((
                    pltpu.async_copy(x_hbm.at[idxl[b].at[0]], rl[b],
                                     sem_g[b]),
                    pltpu.async_copy(x_hbm.at[idxr[b].at[0]], rr[b],
                                     sem_g[b]),
                    pltpu.async_copy(ea_hbm.at[pl.ds(c * CHUNK, CHUNK)],
                                     ea_v[b], sem_e[b]),
                ))
            scatters = []
            for b in range(2):
                gl, gr, ge = gathers[b]
                gl.wait()
                scatters.append(
                    pltpu.async_copy(rl[b], accl_sh.at[dst_c[b].at[0]],
                                     sem_s[b], add=True))
                gr.wait()
                scatters.append(
                    pltpu.async_copy(rr[b], accr_sh.at[dst_c[b].at[0]],
                                     sem_s[b], add=True))
                ge.wait()
                scatters.append(
                    pltpu.async_copy(ea_v[b], acce_sh.at[dst_c[b].at[0]],
                                     sem_s[b], add=True))
            for sc in scatters:
                sc.wait()

        plsc.subcore_barrier()

        r0 = sid * RPT
        o0 = cid * N_ACC + r0
        pltpu.sync_copy(accl_sh.at[pl.ds(r0, RPT)], outl_hbm.at[pl.ds(o0, RPT)])
        pltpu.sync_copy(accr_sh.at[pl.ds(r0, RPT)], outr_hbm.at[pl.ds(o0, RPT)])
        pltpu.sync_copy(acce_sh.at[pl.ds(r0, RPT)], oute_hbm.at[pl.ds(o0, RPT)])

    return k(x2, src2d, dst2d, ea2d)


def _tc_update(pxl, pxr, pe, x, wm1a, wm1b, wm2, ws, b2d):
    R = 1000

    def body(pxl_ref, pxr_ref, pe_ref, x_ref, wm1a_ref, wm1b_ref, wm2_ref,
             ws_ref, b_ref, o_ref):
        sxl = pxl_ref[0] + pxl_ref[1]
        sxr = pxr_ref[0] + pxr_ref[1]
        se = pe_ref[0] + pe_ref[1]
        acc = jnp.dot(sxl, wm1a_ref[...], preferred_element_type=jnp.float32)
        acc += jnp.dot(sxr, wm1b_ref[...], preferred_element_type=jnp.float32)
        acc += jnp.dot(se, wm2_ref[...], preferred_element_type=jnp.float32)
        acc += jnp.dot(x_ref[...], ws_ref[...],
                       preferred_element_type=jnp.float32)
        acc += b_ref[...]
        o_ref[...] = acc * jax.nn.sigmoid(acc)

    return pl.pallas_call(
        body,
        grid=(N // R,),
        in_specs=[
            pl.BlockSpec((NC, R, DH), lambda i: (0, i, 0)),
            pl.BlockSpec((NC, R, DH), lambda i: (0, i, 0)),
            pl.BlockSpec((NC, R, DEP), lambda i: (0, i, 0)),
            pl.BlockSpec((R, D), lambda i: (i, 0)),
            pl.BlockSpec((DH, D), lambda i: (0, 0)),
            pl.BlockSpec((DH, D), lambda i: (0, 0)),
            pl.BlockSpec((DEP, D), lambda i: (0, 0)),
            pl.BlockSpec((D, D), lambda i: (0, 0)),
            pl.BlockSpec((1, D), lambda i: (0, 0)),
        ],
        out_specs=pl.BlockSpec((R, D), lambda i: (i, 0)),
        out_shape=jax.ShapeDtypeStruct((N, D), jnp.float32),
    )(pxl, pxr, pe, x, wm1a, wm1b, wm2, ws, b2d)


def kernel(x, edge_index, edge_attr, W_msg, W_self, b):
    src = edge_index[0]
    dst = edge_index[1]

    pad = E_PAD - E
    src_p = jnp.concatenate([src, jnp.zeros((pad,), jnp.int32)])
    dst_p = jnp.concatenate([dst, jnp.full((pad,), N, jnp.int32)])
    src2d = src_p.reshape(NW * CPT, CHUNK)
    dst2d = dst_p.reshape(NW * CPT, CHUNK)
    ea2d = jnp.pad(edge_attr, ((0, pad), (0, DEP - DE)))
    x2 = x.reshape(2 * N, DH)

    pxl, pxr, pe = _sc_segsum(x2, src2d, dst2d, ea2d)
    pxl = pxl.reshape(NC, N_ACC, DH)[:, :N]
    pxr = pxr.reshape(NC, N_ACC, DH)[:, :N]
    pe = pe.reshape(NC, N_ACC, DEP)[:, :N]

    wm1a = W_msg[:DH]
    wm1b = W_msg[DH:D]
    wm2 = jnp.pad(W_msg[D:], ((0, DEP - DE), (0, 0)))
    b2d = b.reshape(1, D)
    return _tc_update(pxl, pxr, pe, x, wm1a, wm1b, wm2, W_self, b2d)


# speedup vs baseline: 1.0264x; 1.0264x over previous
import functools

import jax
import jax.numpy as jnp
from jax import lax
from jax.experimental import pallas as pl
from jax.experimental.pallas import tpu as pltpu
from jax.experimental.pallas import tpu_sc as plsc

N = 10000
E = 320000
D = 128
DH = 64
DE = 4
DEP = 16

NC = 2
NS = 16
NW = NC * NS
CHUNK = 96
CPT = 106
E_PAD = NW * CPT * CHUNK
N_ACC = 10240
RPT = N_ACC // NS
ZB = 80


def _sc_segsum(x2, src2d, dst2d, ea2d):
    mesh = plsc.VectorSubcoreMesh(core_axis_name="c", subcore_axis_name="s")

    slot_scratch = [
        pltpu.VMEM((1, CHUNK), jnp.int32),
        pltpu.VMEM((1, CHUNK), jnp.int32),
        pltpu.VMEM((1, CHUNK), jnp.int32),
        pltpu.VMEM((1, CHUNK), jnp.int32),
        pltpu.VMEM((CHUNK, DH), jnp.float32),
        pltpu.VMEM((CHUNK, DH), jnp.float32),
        pltpu.VMEM((CHUNK, DEP), jnp.float32),
        pltpu.SemaphoreType.DMA,
        pltpu.SemaphoreType.DMA,
        pltpu.SemaphoreType.DMA,
    ]

    @functools.partial(
        pl.kernel,
        out_type=[
            jax.ShapeDtypeStruct((NC * N_ACC, DH), jnp.float32),
            jax.ShapeDtypeStruct((NC * N_ACC, DH), jnp.float32),
            jax.ShapeDtypeStruct((NC * N_ACC, DEP), jnp.float32),
        ],
        mesh=mesh,
        compiler_params=pltpu.CompilerParams(use_tc_tiling_on_sc=False),
        scratch_types=slot_scratch + slot_scratch + [
            pltpu.VMEM_SHARED((N_ACC, DH), jnp.float32),
            pltpu.VMEM_SHARED((N_ACC, DH), jnp.float32),
            pltpu.VMEM_SHARED((N_ACC, DEP), jnp.float32),
        ],
    )
    def k(x_hbm, src_hbm, dst_hbm, ea_hbm, outl_hbm, outr_hbm, oute_hbm,
          *refs):
        src_c = (refs[0], refs[10])
        dst_c = (refs[1], refs[11])
        idxl = (refs[2], refs[12])
        idxr = (refs[3], refs[13])
        rl = (refs[4], refs[14])
        rr = (refs[5], refs[15])
        ea_v = (refs[6], refs[16])
        sem_g = (refs[7], refs[17])
        sem_e = (refs[8], refs[18])
        sem_s = (refs[9], refs[19])
        accl_sh, accr_sh, acce_sh = refs[20], refs[21], refs[22]

        cid = lax.axis_index("c")
        sid = lax.axis_index("s")
        wid = cid * NS + sid

        zero16 = jnp.zeros((16,), jnp.float32)

        def zx(i, _):
            rl[0][i // (DH // 16), pl.ds((i % (DH // 16)) * 16, 16)] = zero16
            return 0
        lax.fori_loop(0, ZB * (DH // 16), zx, 0)

        def ze(i, _):
            ea_v[0][i, pl.ds(0, 16)] = zero16
            return 0
        lax.fori_loop(0, ZB, ze, 0)

        @pl.loop(0, RPT // ZB)
        def zcopy(i):
            r = sid * RPT + i * ZB
            pltpu.sync_copy(rl[0].at[pl.ds(0, ZB)], accl_sh.at[pl.ds(r, ZB)])
            pltpu.sync_copy(rl[0].at[pl.ds(0, ZB)], accr_sh.at[pl.ds(r, ZB)])
            pltpu.sync_copy(ea_v[0].at[pl.ds(0, ZB)], acce_sh.at[pl.ds(r, ZB)])

        plsc.subcore_barrier()

        base = wid * CPT

        @pl.loop(0, CPT // 2)
        def piter(p):
            c0 = base + 2 * p
            gathers = []
            for b in range(2):
                c = c0 + b
                pltpu.sync_copy(src_hbm.at[pl.ds(c, 1)], src_c[b])
                pltpu.sync_copy(dst_hbm.at[pl.ds(c, 1)], dst_c[b])
                for t in range(CHUNK // 16):
                    s = src_c[b][0, pl.ds(t * 16, 16)]
                    s2 = s + s
                    idxl[b][0, pl.ds(t * 16, 16)] = s2
                    idxr[b][0, pl.ds(t * 16, 16)] = s2 + 1
                gathers.append((
                    pltpu.async_copy(x_hbm.at[idxl[b].at[0]], rl[b],
                                     sem_g[b]),
                    pltpu.async_copy(x_hbm.at[idxr[b].at[0]], rr[b],
                                     sem_g[b]),
                    pltpu.async_copy(ea_hbm.at[pl.ds(c * CHUNK, CHUNK)],
                                     ea_v[b], sem_e[b]),
                ))
            scatters = []
            for b in range(2):
                gl, gr, ge = gathers[b]
                gl.wait()
                scatters.append(
                    pltpu.async_copy(rl[b], accl_sh.at[dst_c[b].at[0]],
                                     sem_s[b], add=True))
                gr.wait()
                scatters.append(
                    pltpu.async_copy(rr[b], accr_sh.at[dst_c[b].at[0]],
                                     sem_s[b], add=True))
                ge.wait()
                scatters.append(
                    pltpu.async_copy(ea_v[b], acce_sh.at[dst_c[b].at[0]],
                                     sem_s[b], add=True))
            for sc in scatters:
                sc.wait()

        plsc.subcore_barrier()

        r0 = sid * RPT
        o0 = cid * N_ACC + r0
        pltpu.sync_copy(accl_sh.at[pl.ds(r0, RPT)], outl_hbm.at[pl.ds(o0, RPT)])
        pltpu.sync_copy(accr_sh.at[pl.ds(r0, RPT)], outr_hbm.at[pl.ds(o0, RPT)])
        pltpu.sync_copy(acce_sh.at[pl.ds(r0, RPT)], oute_hbm.at[pl.ds(o0, RPT)])

    return k(x2, src2d, dst2d, ea2d)


def _tc_update(pxl, pxr, pe, x, wm1a, wm1b, wm2, ws, b2d):
    R = 1000

    def body(pxl_ref, pxr_ref, pe_ref, x_ref, wm1a_ref, wm1b_ref, wm2_ref,
             ws_ref, b_ref, o_ref):
        sxl = pxl_ref[0] + pxl_ref[1]
        sxr = pxr_ref[0] + pxr_ref[1]
        se = pe_ref[0] + pe_ref[1]
        acc = jnp.dot(sxl, wm1a_ref[...], preferred_element_type=jnp.float32)
        acc += jnp.dot(sxr, wm1b_ref[...], preferred_element_type=jnp.float32)
        acc += jnp.dot(se, wm2_ref[...], preferred_element_type=jnp.float32)
        acc += jnp.dot(x_ref[...], ws_ref[...],
                       preferred_element_type=jnp.float32)
        acc += b_ref[...]
        o_ref[...] = acc * jax.nn.sigmoid(acc)

    return pl.pallas_call(
        body,
        grid=(N // R,),
        in_specs=[
            pl.BlockSpec((NC, R, DH), lambda i: (0, i, 0)),
            pl.BlockSpec((NC, R, DH), lambda i: (0, i, 0)),
            pl.BlockSpec((NC, R, DEP), lambda i: (0, i, 0)),
            pl.BlockSpec((R, D), lambda i: (i, 0)),
            pl.BlockSpec((DH, D), lambda i: (0, 0)),
            pl.BlockSpec((DH, D), lambda i: (0, 0)),
            pl.BlockSpec((DEP, D), lambda i: (0, 0)),
            pl.BlockSpec((D, D), lambda i: (0, 0)),
            pl.BlockSpec((1, D), lambda i: (0, 0)),
        ],
        out_specs=pl.BlockSpec((R, D), lambda i: (i, 0)),
        out_shape=jax.ShapeDtypeStruct((N, D), jnp.float32),
    )(pxl, pxr, pe, x, wm1a, wm1b, wm2, ws, b2d)


def kernel(x, edge_index, edge_attr, W_msg, W_self, b):
    src = edge_index[0]
    dst = edge_index[1]

    pad = E_PAD - E
    src_p = jnp.concatenate([src, jnp.zeros((pad,), jnp.int32)])
    dst_p = jnp.concatenate([dst, jnp.full((pad,), N, jnp.int32)])
    src2d = src_p.reshape(NW * CPT, CHUNK)
    dst2d = dst_p.reshape(NW * CPT, CHUNK)
    ea2d = jnp.pad(edge_attr, ((0, pad), (0, DEP - DE)))
    x2 = x.reshape(2 * N, DH)

    pxl, pxr, pe = _sc_segsum(x2, src2d, dst2d, ea2d)
    pxl = pxl.reshape(NC, N_ACC, DH)
    pxr = pxr.reshape(NC, N_ACC, DH)
    pe = pe.reshape(NC, N_ACC, DEP)

    wm1a = W_msg[:DH]
    wm1b = W_msg[DH:D]
    wm2 = jnp.pad(W_msg[D:], ((0, DEP - DE), (0, 0)))
    b2d = b.reshape(1, D)
    return _tc_update(pxl, pxr, pe, x, wm1a, wm1b, wm2, W_self, b2d)
